# baseline (speedup 1.0000x reference)
import jax
import jax.numpy as jnp
from jax.experimental import pallas as pl
from jax.experimental.pallas import tpu as pltpu

_BS, _N, _B, _D, _M, _K = 16, 64, 4, 64, 4096, 8
_NEG = -1e9
_LANES = 128
_NCHUNK = _M // _LANES

_SORT8 = [(0, 1), (2, 3), (4, 5), (6, 7),
          (0, 2), (1, 3), (4, 6), (5, 7),
          (1, 2), (5, 6),
          (0, 4), (1, 5), (2, 6), (3, 7),
          (2, 4), (3, 5),
          (1, 2), (3, 4), (5, 6)]
_CLEAN8 = [(0, 4), (1, 5), (2, 6), (3, 7),
           (0, 2), (1, 3), (4, 6), (5, 7),
           (0, 1), (2, 3), (4, 5), (6, 7)]


def _ce(lst, i, j):
    hi = jnp.maximum(lst[i], lst[j])
    lst[j] = jnp.minimum(lst[i], lst[j])
    lst[i] = hi


def _merge_top8(a, b):
    c = [jnp.maximum(a[i], b[7 - i]) for i in range(8)]
    for (i, j) in _CLEAN8:
        _ce(c, i, j)
    return c


def _masked_scores(q, k, srow):
    scores = jax.lax.dot_general(
        q, k, (((1,), (1,)), ((), ())), preferred_element_type=jnp.float32
    )
    return jnp.where(srow > 0.0, scores, _NEG)


def _topk_thresh(s):
    slices = [s[:, j * _LANES:(j + 1) * _LANES] for j in range(_NCHUNK)]
    groups = []
    for g in range(4):
        grp = slices[g * 8:(g + 1) * 8]
        for (i, j) in _SORT8:
            _ce(grp, i, j)
        groups.append(grp)
    top = _merge_top8(_merge_top8(groups[0], groups[1]),
                      _merge_top8(groups[2], groups[3]))
    top.append(jnp.full_like(top[0], -jnp.inf))

    thr = None
    cnt = None
    row_max = None
    for it in range(_K):
        m = jnp.max(top[0], axis=1, keepdims=True)
        c = jnp.sum(jnp.where(top[0] == m, 1.0, 0.0), axis=1, keepdims=True)
        if it == 0:
            thr = m
            row_max = m
            cnt = c
        else:
            thr = jnp.where(cnt < _K, m, thr)
            cnt = cnt + c
        if it < _K - 1:
            cond = top[0] == m
            for j in range(_K):
                top[j] = jnp.where(cond, top[j + 1], top[j])
    return thr, row_max


def _attend(s, thr, row_max, v):
    e = jnp.where(s >= thr, jnp.exp(s - row_max), 0.0)
    denom = jnp.sum(e, axis=1, keepdims=True)
    out = jax.lax.dot_general(
        e.astype(jnp.bfloat16), v.astype(jnp.bfloat16),
        (((1,), (0,)), ((), ())), preferred_element_type=jnp.float32,
    )
    return out / denom


def _stream_body(q_ref, k_ref, v_ref, s_ref, o_ref):
    ss = [
        _masked_scores(q_ref[0, :, b, :], k_ref[0, b], s_ref[0, b])
        for b in range(_B)
    ]
    tt = [_topk_thresh(ss[b]) for b in range(_B)]
    for b in range(_B):
        thr, row_max = tt[b]
        o_ref[0, :, b, :] = _attend(ss[b], thr, row_max, v_ref[0, b])


@jax.jit
def kernel(q, em_K, em_V, em_S):
    em_S4 = em_S.reshape(_BS, _B, 1, _M)
    grid = (_BS,)
    return pl.pallas_call(
        _stream_body,
        grid=grid,
        in_specs=[
            pl.BlockSpec((1, _N, _B, _D), lambda i: (i, 0, 0, 0)),
            pl.BlockSpec((1, _B, _M, _D), lambda i: (i, 0, 0, 0)),
            pl.BlockSpec((1, _B, _M, _D), lambda i: (i, 0, 0, 0)),
            pl.BlockSpec((1, _B, 1, _M), lambda i: (i, 0, 0, 0)),
        ],
        out_specs=pl.BlockSpec((1, _N, _B, _D), lambda i: (i, 0, 0, 0)),
        out_shape=jax.ShapeDtypeStruct((_BS, _N, _B, _D), jnp.float32),
        compiler_params=pltpu.CompilerParams(
            dimension_semantics=("arbitrary",),
        ),
    )(q, em_K, em_V, em_S4)


# speedup vs baseline: 1.0034x; 1.0034x over previous
import jax
import jax.numpy as jnp
from jax.experimental import pallas as pl
from jax.experimental.pallas import tpu as pltpu

_BS, _N, _B, _D, _M, _K = 16, 64, 4, 64, 4096, 8
_NEG = -1e9
_LANES = 128
_NCHUNK = _M // _LANES

_SORT8 = [(0, 1), (2, 3), (4, 5), (6, 7),
          (0, 2), (1, 3), (4, 6), (5, 7),
          (1, 2), (5, 6),
          (0, 4), (1, 5), (2, 6), (3, 7),
          (2, 4), (3, 5),
          (1, 2), (3, 4), (5, 6)]
_CLEAN8 = [(0, 4), (1, 5), (2, 6), (3, 7),
           (0, 2), (1, 3), (4, 6), (5, 7),
           (0, 1), (2, 3), (4, 5), (6, 7)]


def _ce(lst, i, j):
    hi = jnp.maximum(lst[i], lst[j])
    lst[j] = jnp.minimum(lst[i], lst[j])
    lst[i] = hi


def _merge_top8(a, b):
    c = [jnp.maximum(a[i], b[7 - i]) for i in range(8)]
    for (i, j) in _CLEAN8:
        _ce(c, i, j)
    return c


def _masked_scores(q, k, srow):
    scores = jax.lax.dot_general(
        q, k, (((1,), (1,)), ((), ())), preferred_element_type=jnp.float32
    )
    return jnp.where(srow > 0.0, scores, _NEG)


def _topk_thresh(s):
    slices = [s[:, j * _LANES:(j + 1) * _LANES] for j in range(_NCHUNK)]
    groups = []
    for g in range(4):
        grp = slices[g * 8:(g + 1) * 8]
        for (i, j) in _SORT8:
            _ce(grp, i, j)
        groups.append(grp)
    top = _merge_top8(_merge_top8(groups[0], groups[1]),
                      _merge_top8(groups[2], groups[3]))
    top.append(jnp.full_like(top[0], -jnp.inf))

    thr = None
    cnt = None
    row_max = None
    for it in range(_K):
        m = jnp.max(top[0], axis=1, keepdims=True)
        c = jnp.sum(jnp.where(top[0] == m, 1.0, 0.0), axis=1, keepdims=True)
        if it == 0:
            thr = m
            row_max = m
            cnt = c
        else:
            thr = jnp.where(cnt < _K, m, thr)
            cnt = cnt + c
        if it < _K - 1:
            cond = top[0] == m
            for j in range(_K):
                top[j] = jnp.where(cond, top[j + 1], top[j])
    return thr, row_max


def _attend(s, thr, row_max, v):
    e = jnp.where(s >= thr, jnp.exp(s - row_max), 0.0)
    denom = jnp.sum(e, axis=1, keepdims=True)
    out = jax.lax.dot_general(
        e, v, (((1,), (0,)), ((), ())), preferred_element_type=jnp.float32
    )
    return out / denom


def _stream_body(q_ref, k_ref, v_ref, s_ref, o_ref):
    ss = [
        _masked_scores(q_ref[0, :, b, :], k_ref[0, b], s_ref[0, b])
        for b in range(_B)
    ]
    tt = [_topk_thresh(ss[b]) for b in range(_B)]
    for b in range(_B):
        thr, row_max = tt[b]
        o_ref[0, :, b, :] = _attend(ss[b], thr, row_max, v_ref[0, b])


@jax.jit
def kernel(q, em_K, em_V, em_S):
    em_S4 = em_S.reshape(_BS, _B, 1, _M)
    grid = (_BS,)
    return pl.pallas_call(
        _stream_body,
        grid=grid,
        in_specs=[
            pl.BlockSpec((1, _N, _B, _D), lambda i: (i, 0, 0, 0)),
            pl.BlockSpec((1, _B, _M, _D), lambda i: (i, 0, 0, 0)),
            pl.BlockSpec((1, _B, _M, _D), lambda i: (i, 0, 0, 0)),
            pl.BlockSpec((1, _B, 1, _M), lambda i: (i, 0, 0, 0)),
        ],
        out_specs=pl.BlockSpec((1, _N, _B, _D), lambda i: (i, 0, 0, 0)),
        out_shape=jax.ShapeDtypeStruct((_BS, _N, _B, _D), jnp.float32),
        compiler_params=pltpu.CompilerParams(
            dimension_semantics=("arbitrary",),
        ),
    )(q, em_K, em_V, em_S4)
